# Initial kernel scaffold; baseline (speedup 1.0000x reference)
import jax
import jax.numpy as jnp
from jax.experimental import pallas as pl


def kernel(parent_edge_features, parent_light_edge_features, table):
    raise NotImplementedError("write your pallas kernel here")


# speedup vs baseline: 9.2618x; 9.2618x over previous
import functools

import jax
import jax.numpy as jnp
from jax import lax
from jax.experimental import pallas as pl
from jax.experimental.pallas import tpu as pltpu
from jax.experimental.pallas import tpu_sc as plsc

_NUM_TYPES = 32
_EMB = 16
_E = 1600000
_F = 4
_R = _E * _F
_NC, _NS = 2, 16
_NW = _NC * _NS
_PER_W = _R // _NW
_SUB = 125
_NSUB = 8
_C = _SUB * _NSUB
_CHUNKS = _PER_W // _C

_mesh = plsc.VectorSubcoreMesh(core_axis_name="c", subcore_axis_name="s")


@functools.partial(
    pl.kernel,
    out_type=[
        jax.ShapeDtypeStruct((_R, _EMB), jnp.float32),
        jax.ShapeDtypeStruct((_R, _EMB), jnp.float32),
    ],
    mesh=_mesh,
    scratch_types=[
        pltpu.VMEM((_NUM_TYPES, _EMB), jnp.float32),
        pltpu.VMEM_SHARED((_NUM_TYPES, _EMB), jnp.float32),
        pltpu.VMEM((_NSUB, _SUB), jnp.int32),
        pltpu.VMEM((_C, _EMB), jnp.float32),
        pltpu.SemaphoreType.DMA,
    ],
    compiler_params=pltpu.CompilerParams(use_tc_tiling_on_sc=False),
)
def _encode(pe_idx, ple_idx, table_hbm, pe_out, ple_out,
            table_st, table_sp, idx_v, rows_v, sem):
    cid = lax.axis_index("c")
    sid = lax.axis_index("s")
    wid = sid * _NC + cid

    @pl.when(sid == 0)
    def _stage_table():
        pltpu.sync_copy(table_hbm, table_st)
        pltpu.sync_copy(table_st, table_sp)

    plsc.subcore_barrier()

    base = wid * _PER_W

    def _do(idx_hbm, out_hbm):
        def _chunk(g, carry):
            row0 = pl.multiple_of(base + g * _C, _C)
            pltpu.sync_copy(
                idx_hbm.at[pl.ds(pl.multiple_of(row0 // _SUB, _NSUB), _NSUB)],
                idx_v)
            cps = [
                pltpu.make_async_copy(
                    table_sp.at[idx_v.at[k]],
                    rows_v.at[pl.ds(k * _SUB, _SUB)],
                    sem,
                )
                for k in range(_NSUB)
            ]
            for cp in cps:
                cp.start()
            for cp in cps:
                cp.wait()
            pltpu.sync_copy(rows_v, out_hbm.at[pl.ds(row0, _C)])
            return carry

        lax.fori_loop(0, _CHUNKS, _chunk, 0)

    _do(pe_idx, pe_out)
    _do(ple_idx, ple_out)


def kernel(parent_edge_features, parent_light_edge_features, table):
    pe_idx = parent_edge_features.reshape(_R // _SUB, _SUB)
    ple_idx = parent_light_edge_features.reshape(_R // _SUB, _SUB)
    pe, ple = _encode(pe_idx, ple_idx, table)
    return (pe.reshape(_E, _F * _EMB), ple.reshape(_E, _F * _EMB))


# speedup vs baseline: 17.6540x; 1.9061x over previous
import functools

import jax
import jax.numpy as jnp
from jax import lax
from jax.experimental import pallas as pl
from jax.experimental.pallas import tpu as pltpu
from jax.experimental.pallas import tpu_sc as plsc

_NUM_TYPES = 32
_EMB = 16
_E = 1600000
_F = 4
_NC, _NS = 2, 16
_NW = _NC * _NS
_NT_TOTAL = _E // 128
_NT = 20
_NCHUNK = _NT_TOTAL // _NT
_PAD = 17

_mesh = plsc.VectorSubcoreMesh(core_axis_name="c", subcore_axis_name="s")


@functools.partial(
    pl.kernel,
    out_type=[
        jax.ShapeDtypeStruct((8, _NT_TOTAL, 8, 128), jnp.float32),
        jax.ShapeDtypeStruct((8, _NT_TOTAL, 8, 128), jnp.float32),
    ],
    mesh=_mesh,
    scratch_types=[
        pltpu.VMEM((_NUM_TYPES, _EMB), jnp.float32),
        pltpu.VMEM((_NUM_TYPES, _PAD), jnp.float32),
        pltpu.VMEM((_NT, _F, 128), jnp.int32),
        pltpu.VMEM((_NT, 8, 128), jnp.float32),
    ],
    compiler_params=pltpu.CompilerParams(use_tc_tiling_on_sc=False,
                                         needs_layout_passes=False),
)
def _encode(pe_idx, ple_idx, table_hbm, pe_out, ple_out,
            table_st, table_v, idx_v, out_v):
    cid = lax.axis_index("c")
    sid = lax.axis_index("s")
    wid = sid * _NC + cid

    pltpu.sync_copy(table_hbm, table_st)
    for r in range(_NUM_TYPES):
        table_v[r, pl.ds(0, _EMB)] = table_st[r, pl.ds(0, _EMB)]

    def _do(idx_hbm, out_hbm):
        def _chunk(n, carry):
            c = wid + n * _NW

            @pl.when(c < _NCHUNK)
            def _():
                t0 = pl.multiple_of(c * _NT, _NT)
                pltpu.sync_copy(idx_hbm.at[pl.ds(t0, _NT)], idx_v)
                for h in range(8):
                    f, dhi = h // 2, h % 2

                    def _tile(tt, carry2):
                        for jv in range(8):
                            iv = idx_v[tt, f, pl.ds(jv * 16, 16)]
                            for l in range(8):
                                d = 8 * dhi + l
                                dv = jnp.full((16,), d, jnp.int32)
                                out_v[tt, l, pl.ds(jv * 16, 16)] = (
                                    plsc.load_gather(table_v, [iv, dv]))
                        return carry2

                    lax.fori_loop(0, _NT, _tile, 0)
                    pltpu.sync_copy(out_v, out_hbm.at[h, pl.ds(t0, _NT)])

            return carry

        lax.fori_loop(0, (_NCHUNK + _NW - 1) // _NW, _chunk, 0)

    _do(pe_idx, pe_out)
    _do(ple_idx, ple_out)


def kernel(parent_edge_features, parent_light_edge_features, table):
    def _view_idx(idx):
        return lax.transpose(
            lax.reshape(idx, (_F, _NT_TOTAL, 128), dimensions=(1, 0)),
            (1, 0, 2))

    pe4, ple4 = _encode(_view_idx(parent_edge_features),
                        _view_idx(parent_light_edge_features),
                        table)

    def _view_out(o4):
        return lax.reshape(lax.transpose(o4, (1, 3, 0, 2)),
                           (_E, _F * _EMB))

    return (_view_out(pe4), _view_out(ple4))


# speedup vs baseline: 70.5900x; 3.9985x over previous
import functools

import jax
import jax.numpy as jnp
from jax import lax
from jax.experimental import pallas as pl
from jax.experimental.pallas import tpu as pltpu
from jax.experimental.pallas import tpu_sc as plsc

_NUM_TYPES = 32
_EMB = 16
_E = 1600000
_F = 4
_NC, _NS = 2, 16
_NW = _NC * _NS
_NT_TOTAL = _E // 128
_NT = 20
_NCHUNK = _NT_TOTAL // _NT
_PAD = 17

_mesh = plsc.VectorSubcoreMesh(core_axis_name="c", subcore_axis_name="s")


@functools.partial(
    pl.kernel,
    out_type=[
        jax.ShapeDtypeStruct((8, _NT_TOTAL, 8, 128), jnp.float32),
        jax.ShapeDtypeStruct((8, _NT_TOTAL, 8, 128), jnp.float32),
    ],
    mesh=_mesh,
    scratch_types=[
        pltpu.VMEM((_NUM_TYPES, _EMB), jnp.float32),
        pltpu.VMEM((_NUM_TYPES, _PAD), jnp.float32),
        pltpu.VMEM((_NT, _F, 128), jnp.int32),
        pltpu.VMEM((_NT, 8, 128), jnp.float32),
    ],
    compiler_params=pltpu.CompilerParams(use_tc_tiling_on_sc=False,
                                         needs_layout_passes=False),
)
def _encode(pe_idx, ple_idx, table_hbm, pe_out, ple_out,
            table_st, table_v, idx_v, out_v):
    cid = lax.axis_index("c")
    sid = lax.axis_index("s")
    wid = sid * _NC + cid

    pltpu.sync_copy(table_hbm, table_st)
    for r in range(_NUM_TYPES):
        table_v[r, pl.ds(0, _EMB)] = table_st[r, pl.ds(0, _EMB)]

    def _do(idx_hbm, out_hbm):
        def _chunk(n, carry):
            c = wid + n * _NW

            @pl.when(c < _NCHUNK)
            def _():
                t0 = pl.multiple_of(c * _NT, _NT)
                pltpu.sync_copy(idx_hbm.at[pl.ds(t0, _NT)], idx_v)
                def _h(h, carry2):
                    f = h // 2
                    d0 = 8 * (h % 2)

                    @plsc.parallel_loop(0, _NT, step=1, unroll=2)
                    def _tile(tt):
                        for jv in range(8):
                            iv = idx_v[tt, f, pl.ds(jv * 16, 16)]
                            vals = []
                            for l in range(8):
                                dv = jnp.full((16,), l, jnp.int32) + d0
                                vals.append(
                                    plsc.load_gather(table_v, [iv, dv]))
                            for l in range(8):
                                out_v[tt, l, pl.ds(jv * 16, 16)] = vals[l]

                    pltpu.sync_copy(out_v, out_hbm.at[h, pl.ds(t0, _NT)])
                    return carry2

                lax.fori_loop(0, 8, _h, 0)

            return carry

        lax.fori_loop(0, (_NCHUNK + _NW - 1) // _NW, _chunk, 0)

    _do(pe_idx, pe_out)
    _do(ple_idx, ple_out)


def kernel(parent_edge_features, parent_light_edge_features, table):
    def _view_idx(idx):
        return lax.transpose(
            lax.reshape(idx, (_F, _NT_TOTAL, 128), dimensions=(1, 0)),
            (1, 0, 2))

    pe4, ple4 = _encode(_view_idx(parent_edge_features),
                        _view_idx(parent_light_edge_features),
                        table)

    def _view_out(o4):
        return lax.reshape(lax.transpose(o4, (1, 3, 0, 2)),
                           (_E, _F * _EMB))

    return (_view_out(pe4), _view_out(ple4))


# speedup vs baseline: 73.3695x; 1.0394x over previous
import functools

import jax
import jax.numpy as jnp
from jax import lax
from jax.experimental import pallas as pl
from jax.experimental.pallas import tpu as pltpu
from jax.experimental.pallas import tpu_sc as plsc

_NUM_TYPES = 32
_EMB = 16
_E = 1600000
_F = 4
_NC, _NS = 2, 16
_NW = _NC * _NS
_NT_TOTAL = _E // 128
_NT = 20
_NCHUNK = _NT_TOTAL // _NT
_PAD = 17

_mesh = plsc.VectorSubcoreMesh(core_axis_name="c", subcore_axis_name="s")


@functools.partial(
    pl.kernel,
    out_type=[
        jax.ShapeDtypeStruct((8, _NT_TOTAL, 8, 128), jnp.float32),
        jax.ShapeDtypeStruct((8, _NT_TOTAL, 8, 128), jnp.float32),
    ],
    mesh=_mesh,
    scratch_types=[
        pltpu.VMEM((_NUM_TYPES, _EMB), jnp.float32),
        pltpu.VMEM((_NUM_TYPES, _PAD), jnp.float32),
        pltpu.VMEM((_NT, _F, 128), jnp.int32),
        pltpu.VMEM((2, _NT, 8, 128), jnp.float32),
        pltpu.SemaphoreType.DMA,
        pltpu.SemaphoreType.DMA,
    ],
    compiler_params=pltpu.CompilerParams(use_tc_tiling_on_sc=False,
                                         needs_layout_passes=False),
)
def _encode(pe_idx, ple_idx, table_hbm, pe_out, ple_out,
            table_st, table_v, idx_v, out_v, sem0, sem1):
    cid = lax.axis_index("c")
    sid = lax.axis_index("s")
    wid = sid * _NC + cid

    pltpu.sync_copy(table_hbm, table_st)
    for r in range(_NUM_TYPES):
        table_v[r, pl.ds(0, _EMB)] = table_st[r, pl.ds(0, _EMB)]

    def _do(idx_hbm, out_hbm):
        def _chunk(n, carry):
            c = wid + n * _NW

            @pl.when(c < _NCHUNK)
            def _():
                t0 = pl.multiple_of(c * _NT, _NT)
                pltpu.sync_copy(idx_hbm.at[pl.ds(t0, _NT)], idx_v)
                sems = (sem0, sem1)

                def _hh(hh, carry2):
                    for p in range(2):
                        h = 2 * hh + p

                        @pl.when(hh >= 1)
                        def _wait():
                            pltpu.make_async_copy(
                                out_v.at[p],
                                out_hbm.at[h - 2, pl.ds(t0, _NT)],
                                sems[p]).wait()

                        @plsc.parallel_loop(0, _NT, step=1, unroll=2)
                        def _tile(tt):
                            for jv in range(8):
                                iv = idx_v[tt, hh, pl.ds(jv * 16, 16)]
                                vals = []
                                for l in range(8):
                                    dv = jnp.full((16,), 8 * p + l, jnp.int32)
                                    vals.append(
                                        plsc.load_gather(table_v, [iv, dv]))
                                for l in range(8):
                                    out_v[p, tt, l, pl.ds(jv * 16, 16)] = (
                                        vals[l])

                        pltpu.make_async_copy(
                            out_v.at[p],
                            out_hbm.at[h, pl.ds(t0, _NT)],
                            sems[p]).start()
                    return carry2

                lax.fori_loop(0, 4, _hh, 0)
                for p in range(2):
                    pltpu.make_async_copy(
                        out_v.at[p],
                        out_hbm.at[6 + p, pl.ds(t0, _NT)],
                        sems[p]).wait()

            return carry

        lax.fori_loop(0, (_NCHUNK + _NW - 1) // _NW, _chunk, 0)

    _do(pe_idx, pe_out)
    _do(ple_idx, ple_out)


def kernel(parent_edge_features, parent_light_edge_features, table):
    def _view_idx(idx):
        return lax.transpose(
            lax.reshape(idx, (_F, _NT_TOTAL, 128), dimensions=(1, 0)),
            (1, 0, 2))

    pe4, ple4 = _encode(_view_idx(parent_edge_features),
                        _view_idx(parent_light_edge_features),
                        table)

    def _view_out(o4):
        return lax.reshape(lax.transpose(o4, (1, 3, 0, 2)),
                           (_E, _F * _EMB))

    return (_view_out(pe4), _view_out(ple4))


# speedup vs baseline: 83.4709x; 1.1377x over previous
import functools

import jax
import jax.numpy as jnp
from jax import lax
from jax.experimental import pallas as pl
from jax.experimental.pallas import tpu as pltpu
from jax.experimental.pallas import tpu_sc as plsc

_NUM_TYPES = 32
_EMB = 16
_E = 1600000
_F = 4
_NC, _NS = 2, 16
_NW = _NC * _NS
_NT_TOTAL = _E // 128
_NT = 20
_NCHUNK = _NT_TOTAL // _NT
_PAD = 17

_mesh = plsc.VectorSubcoreMesh(core_axis_name="c", subcore_axis_name="s")


@functools.partial(
    pl.kernel,
    out_type=[
        jax.ShapeDtypeStruct((8, _NT_TOTAL, 8, 128), jnp.float32),
        jax.ShapeDtypeStruct((8, _NT_TOTAL, 8, 128), jnp.float32),
    ],
    mesh=_mesh,
    scratch_types=[
        pltpu.VMEM((_NUM_TYPES, _EMB), jnp.float32),
        pltpu.VMEM((_NUM_TYPES, _PAD), jnp.float32),
        pltpu.VMEM((_NT, _F, 128), jnp.int32),
        pltpu.VMEM((2, _NT, 8, 128), jnp.float32),
        pltpu.SemaphoreType.DMA,
        pltpu.SemaphoreType.DMA,
    ],
    compiler_params=pltpu.CompilerParams(use_tc_tiling_on_sc=False,
                                         needs_layout_passes=False),
)
def _encode(pe_idx, ple_idx, table_hbm, pe_out, ple_out,
            table_st, table_v, idx_v, out_v, sem0, sem1):
    cid = lax.axis_index("c")
    sid = lax.axis_index("s")
    wid = sid * _NC + cid

    pltpu.sync_copy(table_hbm, table_st)
    for r in range(_NUM_TYPES):
        table_v[r, pl.ds(0, _EMB)] = table_st[r, pl.ds(0, _EMB)]

    def _do(idx_hbm, out_hbm):
        def _chunk(n, carry):
            c = wid + n * _NW

            @pl.when(c < _NCHUNK)
            def _():
                t0 = pl.multiple_of(c * _NT, _NT)
                pltpu.sync_copy(idx_hbm.at[pl.ds(t0, _NT)], idx_v)
                sems = (sem0, sem1)

                def _hh(hh, carry2):
                    for p in range(2):
                        h = 2 * hh + p

                        @pl.when(hh >= 1)
                        def _wait():
                            pltpu.make_async_copy(
                                out_v.at[p],
                                out_hbm.at[h - 2, pl.ds(t0, _NT)],
                                sems[p]).wait()

                        @plsc.parallel_loop(0, _NT, step=1, unroll=4)
                        def _tile(tt):
                            for jv in range(8):
                                iv = idx_v[tt, hh, pl.ds(jv * 16, 16)]
                                vals = []
                                for l in range(8):
                                    dv = jnp.full((16,), 8 * p + l, jnp.int32)
                                    vals.append(
                                        plsc.load_gather(table_v, [iv, dv]))
                                for l in range(8):
                                    out_v[p, tt, l, pl.ds(jv * 16, 16)] = (
                                        vals[l])

                        pltpu.make_async_copy(
                            out_v.at[p],
                            out_hbm.at[h, pl.ds(t0, _NT)],
                            sems[p]).start()
                    return carry2

                lax.fori_loop(0, 4, _hh, 0)
                for p in range(2):
                    pltpu.make_async_copy(
                        out_v.at[p],
                        out_hbm.at[6 + p, pl.ds(t0, _NT)],
                        sems[p]).wait()

            return carry

        lax.fori_loop(0, (_NCHUNK + _NW - 1) // _NW, _chunk, 0)

    _do(pe_idx, pe_out)
    _do(ple_idx, ple_out)


def kernel(parent_edge_features, parent_light_edge_features, table):
    def _view_idx(idx):
        return lax.transpose(
            lax.reshape(idx, (_F, _NT_TOTAL, 128), dimensions=(1, 0)),
            (1, 0, 2))

    pe4, ple4 = _encode(_view_idx(parent_edge_features),
                        _view_idx(parent_light_edge_features),
                        table)

    def _view_out(o4):
        return lax.reshape(lax.transpose(o4, (1, 3, 0, 2)),
                           (_E, _F * _EMB))

    return (_view_out(pe4), _view_out(ple4))
